# Initial kernel scaffold; baseline (speedup 1.0000x reference)
import jax
import jax.numpy as jnp
from jax.experimental import pallas as pl


def kernel(x_X, eattr_X, x_Y, eattr_Y, noise, g1_w1, g1_w2, g1_w3, g1_b, g2_u1, g2_u2, g2_b, w_mu, b_mu, w_lv, b_lv, w3, w4, b2, edge_index_X, edge_index_Y, gid_X, gid_Y):
    raise NotImplementedError("write your pallas kernel here")


# speedup vs baseline: 1.6059x; 1.6059x over previous
import functools

import jax
import jax.numpy as jnp
from jax import lax
from jax.experimental import pallas as pl
from jax.experimental.pallas import tpu as pltpu
from jax.experimental.pallas import tpu_sc as plsc

F32 = jnp.float32
NC, NS = 2, 16
NW = NC * NS

N1 = 10000
E1 = 320000
EP1 = 327680
EP = 2 * EP1
NPAD = 10240
SEGP = 128


GROUP = 8
OP = 128
HALF = 512


def _sc_gather(D, B, n_groups, active_wids):
    mesh = plsc.VectorSubcoreMesh(core_axis_name="c", subcore_axis_name="s")

    @functools.partial(
        pl.kernel, mesh=mesh,
        out_type=jax.ShapeDtypeStruct((B, D), F32),
        scratch_types=[
            pltpu.VMEM((GROUP, OP), jnp.int32),
            pltpu.VMEM((HALF, D), F32),
            pltpu.SemaphoreType.DMA,
        ],
    )
    def k(table_hbm, idx_hbm, out_hbm, idx_v, rows_v, sem):
        wid = lax.axis_index("s") * NC + lax.axis_index("c")
        idx_row0 = wid * (n_groups * GROUP)

        def body(j, carry):
            r0 = idx_row0 + j * GROUP
            pltpu.sync_copy(idx_hbm.at[pl.ds(r0, GROUP)], idx_v)
            for half in range(2):
                cps = [
                    pltpu.async_copy(table_hbm.at[idx_v.at[half * 4 + kk]],
                                     rows_v.at[pl.ds(kk * OP, OP)], sem)
                    for kk in range(4)
                ]
                for c in cps:
                    c.wait()
                pltpu.sync_copy(
                    rows_v, out_hbm.at[pl.ds((r0 + half * 4) * OP, HALF)])
            return carry

        @pl.when(wid < active_wids)
        def _():
            lax.fori_loop(0, n_groups, body, 0)

    return k


def _sc_segsum(B, D, npad, n_groups, active_sids):
    zr = npad // NS
    qr = HALF // 2
    mesh = plsc.VectorSubcoreMesh(core_axis_name="c", subcore_axis_name="s")

    @functools.partial(
        pl.kernel, mesh=mesh,
        out_type=jax.ShapeDtypeStruct((NC * npad, D), F32),
        scratch_types=[
            pltpu.VMEM((GROUP, OP), jnp.int32),
            pltpu.VMEM((qr, D), F32),
            pltpu.VMEM_SHARED((npad, D), F32),
        ],
    )
    def k(vals_hbm, idx_hbm, zeros_hbm, out_hbm, idx_v, val_v, acc_sh):
        cid = lax.axis_index("c")
        sid = lax.axis_index("s")
        pltpu.sync_copy(zeros_hbm.at[pl.ds(sid * zr, zr)],
                        acc_sh.at[pl.ds(sid * zr, zr)])
        plsc.subcore_barrier()
        idx_row0 = (cid * active_sids + sid) * (n_groups * GROUP)

        def body(j, carry):
            r0 = idx_row0 + j * GROUP
            pltpu.sync_copy(idx_hbm.at[pl.ds(r0, GROUP)], idx_v)
            for q in range(4):
                pltpu.sync_copy(
                    vals_hbm.at[pl.ds((r0 + q * 2) * OP, qr)], val_v)
                for kk in range(2):
                    pltpu.sync_copy(val_v.at[pl.ds(kk * OP, OP)],
                                    acc_sh.at[idx_v.at[q * 2 + kk]],
                                    add=True)
            return carry

        @pl.when(sid < active_sids)
        def _():
            lax.fori_loop(0, n_groups, body, 0)

        plsc.subcore_barrier()
        pltpu.sync_copy(acc_sh.at[pl.ds(sid * zr, zr)],
                        out_hbm.at[pl.ds(cid * npad + sid * zr, zr)])

    return k


_gather_edges = _sc_gather(128, EP, 20, 32)
_gather_z = _sc_gather(128, NPAD, 1, 10)
_segsum_edges = _sc_segsum(EP, 128, NPAD, 20, 16)
_segsum_nodes = _sc_segsum(2 * NPAD, 128, SEGP, 1, 10)


def _mm(x, w, bm):
    M, K = x.shape
    N = w.shape[1]

    def body(x_ref, w_ref, o_ref):
        o_ref[...] = jnp.dot(x_ref[...], w_ref[...],
                             preferred_element_type=F32)

    return pl.pallas_call(
        body, grid=(M // bm,),
        in_specs=[pl.BlockSpec((bm, K), lambda i: (i, 0)),
                  pl.BlockSpec((K, N), lambda i: (0, 0))],
        out_specs=pl.BlockSpec((bm, N), lambda i: (i, 0)),
        out_shape=jax.ShapeDtypeStruct((M, N), F32),
    )(x, w)


def _edge_base(xs, ecat, w2, b):
    bm = 2048

    def body(xs_ref, ea_ref, w2_ref, b_ref, base_ref, msg_ref):
        base = xs_ref[...] + jnp.dot(ea_ref[...], w2_ref[...],
                                     preferred_element_type=F32) + b_ref[...]
        base_ref[...] = base
        msg_ref[...] = jnp.maximum(base, 0.0)

    return pl.pallas_call(
        body, grid=(EP // bm,),
        in_specs=[pl.BlockSpec((bm, 128), lambda i: (i, 0)),
                  pl.BlockSpec((bm, 16), lambda i: (i, 0)),
                  pl.BlockSpec((16, 128), lambda i: (0, 0)),
                  pl.BlockSpec((1, 128), lambda i: (0, 0))],
        out_specs=[pl.BlockSpec((bm, 128), lambda i: (i, 0)),
                   pl.BlockSpec((bm, 128), lambda i: (i, 0))],
        out_shape=[jax.ShapeDtypeStruct((EP, 128), F32),
                   jax.ShapeDtypeStruct((EP, 128), F32)],
    )(xs, ecat, w2, b)


def _relu_add(base, g):
    bm = 2048

    def body(b_ref, g_ref, o_ref):
        o_ref[...] = jnp.maximum(b_ref[...] + g_ref[...], 0.0)

    return pl.pallas_call(
        body, grid=(EP // bm,),
        in_specs=[pl.BlockSpec((bm, 128), lambda i: (i, 0)),
                  pl.BlockSpec((bm, 128), lambda i: (i, 0))],
        out_specs=pl.BlockSpec((bm, 128), lambda i: (i, 0)),
        out_shape=jax.ShapeDtypeStruct((EP, 128), F32),
    )(base, g)


def _readout(xcat, s3d, u1, u2, b):
    bm = 400

    def body(x_ref, s_ref, u1_ref, u2_ref, b_ref, o_ref):
        h = (jnp.dot(x_ref[...], u1_ref[...], preferred_element_type=F32)
             + jnp.dot(s_ref[0], u2_ref[...], preferred_element_type=F32)
             + b_ref[...])
        o_ref[0] = jnp.maximum(h, 0.0)

    return pl.pallas_call(
        body, grid=(2 * N1 // bm,),
        in_specs=[pl.BlockSpec((bm, 128), lambda i: (i, 0)),
                  pl.BlockSpec((1, bm, 128), lambda i: (i // 25, i % 25, 0)),
                  pl.BlockSpec((128, 128), lambda i: (0, 0)),
                  pl.BlockSpec((128, 128), lambda i: (0, 0)),
                  pl.BlockSpec((1, 128), lambda i: (0, 0))],
        out_specs=pl.BlockSpec((1, bm, 128), lambda i: (i // 25, i % 25, 0)),
        out_shape=jax.ShapeDtypeStruct((2, NPAD, 128), F32),
    )(xcat, s3d, u1, u2, b)


def _head(p3d, noise_pad, w_mu, b_mu, w_lv, b_lv):
    def body(p_ref, n_ref, wm_ref, bm_ref, wl_ref, bl_ref, z_ref, kl_ref):
        p = p_ref[...]
        delta = p[0, :64] - p[1, :64]
        mu = jnp.dot(delta, wm_ref[...], preferred_element_type=F32) + bm_ref[...]
        lv = jnp.dot(delta, wl_ref[...], preferred_element_type=F32) + bl_ref[...]
        z = jnp.exp(0.5 * lv) * (mu + n_ref[...])
        mask = lax.broadcasted_iota(jnp.int32, (64, 64), 0) < 50
        zm = jnp.where(mask, z, 0.0)
        z_ref[...] = jnp.concatenate([zm, jnp.zeros((64, 64), F32)], axis=1)
        t = jnp.where(mask, 1.0 + lv - mu * mu - jnp.exp(lv), 0.0)
        kl_ref[...] = jnp.full((8, 128), -0.5 * jnp.sum(t) / 50.0, F32)

    return pl.pallas_call(
        body, grid=(1,),
        in_specs=[pl.BlockSpec((2, SEGP, 128), lambda i: (0, 0, 0)),
                  pl.BlockSpec((64, 64), lambda i: (0, 0)),
                  pl.BlockSpec((128, 64), lambda i: (0, 0)),
                  pl.BlockSpec((1, 64), lambda i: (0, 0)),
                  pl.BlockSpec((128, 64), lambda i: (0, 0)),
                  pl.BlockSpec((1, 64), lambda i: (0, 0))],
        out_specs=[pl.BlockSpec((64, 128), lambda i: (0, 0)),
                   pl.BlockSpec((8, 128), lambda i: (0, 0))],
        out_shape=[jax.ShapeDtypeStruct((64, 128), F32),
                   jax.ShapeDtypeStruct((8, 128), F32)],
    )(p3d, noise_pad, w_mu, b_mu, w_lv, b_lv)


def _decoder(hx, zn, w3, w4, b2):
    bm = 1000

    def body(h_ref, z_ref, w3_ref, w4_ref, b_ref, o_ref):
        o_ref[...] = jnp.maximum(
            jnp.dot(h_ref[...], w3_ref[...], preferred_element_type=F32)
            + jnp.dot(z_ref[...][:, :64], w4_ref[...],
                      preferred_element_type=F32)
            + b_ref[...], 0.0)

    return pl.pallas_call(
        body, grid=(N1 // bm,),
        in_specs=[pl.BlockSpec((bm, 128), lambda i: (i, 0)),
                  pl.BlockSpec((bm, 128), lambda i: (i, 0)),
                  pl.BlockSpec((128, 128), lambda i: (0, 0)),
                  pl.BlockSpec((64, 128), lambda i: (0, 0)),
                  pl.BlockSpec((1, 128), lambda i: (0, 0))],
        out_specs=pl.BlockSpec((bm, 128), lambda i: (i, 0)),
        out_shape=jax.ShapeDtypeStruct((N1, 128), F32),
    )(hx, zn, w3, w4, b2)


def kernel(x_X, eattr_X, x_Y, eattr_Y, noise, g1_w1, g1_w2, g1_w3, g1_b,
           g2_u1, g2_u2, g2_b, w_mu, b_mu, w_lv, b_lv, w3, w4, b2,
           edge_index_X, edge_index_Y, gid_X, gid_Y):
    pad_e = EP1 - E1
    zpad = jnp.zeros((pad_e,), jnp.int32)
    trash = jnp.full((pad_e,), NPAD - 1, jnp.int32)
    srcX, dstX = edge_index_X[0], edge_index_X[1]
    srcY, dstY = edge_index_Y[0], edge_index_Y[1]
    src_xe = jnp.concatenate([srcX, zpad, srcY + N1, zpad]).reshape(-1, 128)
    src_a = jnp.concatenate([srcX, zpad, srcY + NPAD, zpad]).reshape(-1, 128)
    dst = jnp.concatenate([dstX, trash, dstY, trash]).reshape(-1, 128)
    gpad = jnp.full((NPAD - N1,), SEGP - 1, jnp.int32)
    gid_cat = jnp.concatenate([gid_X, gpad, gid_Y, gpad]).reshape(-1, 128)
    gid_zidx = jnp.concatenate(
        [gid_X, jnp.zeros((NPAD - N1,), jnp.int32)]).reshape(-1, 128)

    zeros_node = jnp.zeros((NPAD, 128), F32)
    zeros_seg = jnp.zeros((SEGP, 128), F32)
    xcat = jnp.concatenate([x_X, x_Y])
    epad = jnp.zeros((pad_e, 16), F32)
    ecat = jnp.concatenate([eattr_X, epad, eattr_Y, epad])
    noise_pad = jnp.pad(noise, ((0, 14), (0, 0)))

    xe = _mm(xcat, g1_w1, 1000)
    xs = _gather_edges(xe, src_xe)
    base, msg = _edge_base(xs, ecat, g1_w2, g1_b)
    s = None
    for it in range(3):
        s = _segsum_edges(msg, dst, zeros_node)
        if it < 2:
            a = _mm(s, g1_w3, 1024)
            g = _gather_edges(a, src_a)
            msg = _relu_add(base, g)

    h = _readout(xcat, s.reshape(2, NPAD, 128), g2_u1, g2_u2, g2_b)
    hp = h.reshape(2 * NPAD, 128)
    p = _segsum_nodes(hp, gid_cat, zeros_seg)
    z, klb = _head(p.reshape(2, SEGP, 128), noise_pad, w_mu,
                   b_mu.reshape(1, 64), w_lv, b_lv.reshape(1, 64))
    zn = _gather_z(z, gid_zidx)
    x_tilde = _decoder(hp[:N1], zn[:N1], w3, w4, b2)
    return (x_tilde, klb[0, 0])


# speedup vs baseline: 1.6844x; 1.0489x over previous
import functools

import jax
import jax.numpy as jnp
from jax import lax
from jax.experimental import pallas as pl
from jax.experimental.pallas import tpu as pltpu
from jax.experimental.pallas import tpu_sc as plsc

F32 = jnp.float32
NC, NS = 2, 16
NW = NC * NS

N1 = 10000
E1 = 320000
EP1 = 327680
EP = 2 * EP1
NPAD = 10240
SEGP = 128


GROUP = 8
OP = 128
HALF = 512


def _sc_gather(D, B, n_groups, active_wids):
    mesh = plsc.VectorSubcoreMesh(core_axis_name="c", subcore_axis_name="s")

    @functools.partial(
        pl.kernel, mesh=mesh,
        out_type=jax.ShapeDtypeStruct((B, D), F32),
        scratch_types=[
            pltpu.VMEM((GROUP, OP), jnp.int32),
            pltpu.VMEM((HALF, D), F32),
            pltpu.SemaphoreType.DMA,
        ],
    )
    def k(table_hbm, idx_hbm, out_hbm, idx_v, rows_v, sem):
        wid = lax.axis_index("s") * NC + lax.axis_index("c")
        idx_row0 = wid * (n_groups * GROUP)

        def body(j, carry):
            r0 = idx_row0 + j * GROUP
            pltpu.sync_copy(idx_hbm.at[pl.ds(r0, GROUP)], idx_v)
            for half in range(2):
                cps = [
                    pltpu.async_copy(table_hbm.at[idx_v.at[half * 4 + kk]],
                                     rows_v.at[pl.ds(kk * OP, OP)], sem)
                    for kk in range(4)
                ]
                for c in cps:
                    c.wait()
                pltpu.sync_copy(
                    rows_v, out_hbm.at[pl.ds((r0 + half * 4) * OP, HALF)])
            return carry

        @pl.when(wid < active_wids)
        def _():
            lax.fori_loop(0, n_groups, body, 0)

    return k


def _sc_segsum(B, D, npad, n_groups, active_sids):
    zr = npad // NS
    mesh = plsc.VectorSubcoreMesh(core_axis_name="c", subcore_axis_name="s")

    @functools.partial(
        pl.kernel, mesh=mesh,
        out_type=jax.ShapeDtypeStruct((NC * npad, D), F32),
        scratch_types=[
            pltpu.VMEM((GROUP, OP), jnp.int32),
            pltpu.VMEM((OP, D), F32),
            pltpu.VMEM((OP, D), F32),
            pltpu.SemaphoreType.DMA,
            pltpu.SemaphoreType.DMA,
            pltpu.VMEM_SHARED((npad, D), F32),
        ],
    )
    def k(vals_hbm, idx_hbm, zeros_hbm, out_hbm, idx_v, vb0, vb1,
          sem_l, sem_s, acc_sh):
        cid = lax.axis_index("c")
        sid = lax.axis_index("s")
        pltpu.sync_copy(zeros_hbm.at[pl.ds(sid * zr, zr)],
                        acc_sh.at[pl.ds(sid * zr, zr)])
        plsc.subcore_barrier()
        idx_row0 = (cid * active_sids + sid) * (n_groups * GROUP)

        vbufs = (vb0, vb1)

        def body(j, carry):
            r0 = idx_row0 + j * GROUP
            pltpu.sync_copy(idx_hbm.at[pl.ds(r0, GROUP)], idx_v)
            loads = {}
            scats = {}
            for kk in range(2):
                loads[kk] = pltpu.async_copy(
                    vals_hbm.at[pl.ds((r0 + kk) * OP, OP)], vbufs[kk], sem_l)
            for kk in range(GROUP):
                loads[kk].wait()
                scats[kk] = pltpu.async_copy(
                    vbufs[kk % 2], acc_sh.at[idx_v.at[kk]], sem_s, add=True)
                if kk + 2 < GROUP:
                    scats[kk].wait()
                    loads[kk + 2] = pltpu.async_copy(
                        vals_hbm.at[pl.ds((r0 + kk + 2) * OP, OP)],
                        vbufs[kk % 2], sem_l)
            scats[GROUP - 2].wait()
            scats[GROUP - 1].wait()
            return carry

        @pl.when(sid < active_sids)
        def _():
            lax.fori_loop(0, n_groups, body, 0)

        plsc.subcore_barrier()
        pltpu.sync_copy(acc_sh.at[pl.ds(sid * zr, zr)],
                        out_hbm.at[pl.ds(cid * npad + sid * zr, zr)])

    return k


_gather_edges = _sc_gather(128, EP, 20, 32)
_gather_z = _sc_gather(128, NPAD, 1, 10)
_segsum_edges = _sc_segsum(EP, 128, NPAD, 20, 16)
_segsum_nodes = _sc_segsum(2 * NPAD, 128, SEGP, 1, 10)


def _mm(x, w, bm):
    M, K = x.shape
    N = w.shape[1]

    def body(x_ref, w_ref, o_ref):
        o_ref[...] = jnp.dot(x_ref[...], w_ref[...],
                             preferred_element_type=F32)

    return pl.pallas_call(
        body, grid=(M // bm,),
        in_specs=[pl.BlockSpec((bm, K), lambda i: (i, 0)),
                  pl.BlockSpec((K, N), lambda i: (0, 0))],
        out_specs=pl.BlockSpec((bm, N), lambda i: (i, 0)),
        out_shape=jax.ShapeDtypeStruct((M, N), F32),
    )(x, w)


def _edge_base(xs, ecat, w2, b):
    bm = 2048

    def body(xs_ref, ea_ref, w2_ref, b_ref, base_ref, msg_ref):
        base = xs_ref[...] + jnp.dot(ea_ref[...], w2_ref[...],
                                     preferred_element_type=F32) + b_ref[...]
        base_ref[...] = base
        msg_ref[...] = jnp.maximum(base, 0.0)

    return pl.pallas_call(
        body, grid=(EP // bm,),
        in_specs=[pl.BlockSpec((bm, 128), lambda i: (i, 0)),
                  pl.BlockSpec((bm, 16), lambda i: (i, 0)),
                  pl.BlockSpec((16, 128), lambda i: (0, 0)),
                  pl.BlockSpec((1, 128), lambda i: (0, 0))],
        out_specs=[pl.BlockSpec((bm, 128), lambda i: (i, 0)),
                   pl.BlockSpec((bm, 128), lambda i: (i, 0))],
        out_shape=[jax.ShapeDtypeStruct((EP, 128), F32),
                   jax.ShapeDtypeStruct((EP, 128), F32)],
    )(xs, ecat, w2, b)


def _relu_add(base, g):
    bm = 2048

    def body(b_ref, g_ref, o_ref):
        o_ref[...] = jnp.maximum(b_ref[...] + g_ref[...], 0.0)

    return pl.pallas_call(
        body, grid=(EP // bm,),
        in_specs=[pl.BlockSpec((bm, 128), lambda i: (i, 0)),
                  pl.BlockSpec((bm, 128), lambda i: (i, 0))],
        out_specs=pl.BlockSpec((bm, 128), lambda i: (i, 0)),
        out_shape=jax.ShapeDtypeStruct((EP, 128), F32),
    )(base, g)


def _readout(xcat, s3d, u1, u2, b):
    bm = 400

    def body(x_ref, s_ref, u1_ref, u2_ref, b_ref, o_ref):
        h = (jnp.dot(x_ref[...], u1_ref[...], preferred_element_type=F32)
             + jnp.dot(s_ref[0], u2_ref[...], preferred_element_type=F32)
             + b_ref[...])
        o_ref[0] = jnp.maximum(h, 0.0)

    return pl.pallas_call(
        body, grid=(2 * N1 // bm,),
        in_specs=[pl.BlockSpec((bm, 128), lambda i: (i, 0)),
                  pl.BlockSpec((1, bm, 128), lambda i: (i // 25, i % 25, 0)),
                  pl.BlockSpec((128, 128), lambda i: (0, 0)),
                  pl.BlockSpec((128, 128), lambda i: (0, 0)),
                  pl.BlockSpec((1, 128), lambda i: (0, 0))],
        out_specs=pl.BlockSpec((1, bm, 128), lambda i: (i // 25, i % 25, 0)),
        out_shape=jax.ShapeDtypeStruct((2, NPAD, 128), F32),
    )(xcat, s3d, u1, u2, b)


def _head(p3d, noise_pad, w_mu, b_mu, w_lv, b_lv):
    def body(p_ref, n_ref, wm_ref, bm_ref, wl_ref, bl_ref, z_ref, kl_ref):
        p = p_ref[...]
        delta = p[0, :64] - p[1, :64]
        mu = jnp.dot(delta, wm_ref[...], preferred_element_type=F32) + bm_ref[...]
        lv = jnp.dot(delta, wl_ref[...], preferred_element_type=F32) + bl_ref[...]
        z = jnp.exp(0.5 * lv) * (mu + n_ref[...])
        mask = lax.broadcasted_iota(jnp.int32, (64, 64), 0) < 50
        zm = jnp.where(mask, z, 0.0)
        z_ref[...] = jnp.concatenate([zm, jnp.zeros((64, 64), F32)], axis=1)
        t = jnp.where(mask, 1.0 + lv - mu * mu - jnp.exp(lv), 0.0)
        kl_ref[...] = jnp.full((8, 128), -0.5 * jnp.sum(t) / 50.0, F32)

    return pl.pallas_call(
        body, grid=(1,),
        in_specs=[pl.BlockSpec((2, SEGP, 128), lambda i: (0, 0, 0)),
                  pl.BlockSpec((64, 64), lambda i: (0, 0)),
                  pl.BlockSpec((128, 64), lambda i: (0, 0)),
                  pl.BlockSpec((1, 64), lambda i: (0, 0)),
                  pl.BlockSpec((128, 64), lambda i: (0, 0)),
                  pl.BlockSpec((1, 64), lambda i: (0, 0))],
        out_specs=[pl.BlockSpec((64, 128), lambda i: (0, 0)),
                   pl.BlockSpec((8, 128), lambda i: (0, 0))],
        out_shape=[jax.ShapeDtypeStruct((64, 128), F32),
                   jax.ShapeDtypeStruct((8, 128), F32)],
    )(p3d, noise_pad, w_mu, b_mu, w_lv, b_lv)


def _decoder(hx, zn, w3, w4, b2):
    bm = 1000

    def body(h_ref, z_ref, w3_ref, w4_ref, b_ref, o_ref):
        o_ref[...] = jnp.maximum(
            jnp.dot(h_ref[...], w3_ref[...], preferred_element_type=F32)
            + jnp.dot(z_ref[...][:, :64], w4_ref[...],
                      preferred_element_type=F32)
            + b_ref[...], 0.0)

    return pl.pallas_call(
        body, grid=(N1 // bm,),
        in_specs=[pl.BlockSpec((bm, 128), lambda i: (i, 0)),
                  pl.BlockSpec((bm, 128), lambda i: (i, 0)),
                  pl.BlockSpec((128, 128), lambda i: (0, 0)),
                  pl.BlockSpec((64, 128), lambda i: (0, 0)),
                  pl.BlockSpec((1, 128), lambda i: (0, 0))],
        out_specs=pl.BlockSpec((bm, 128), lambda i: (i, 0)),
        out_shape=jax.ShapeDtypeStruct((N1, 128), F32),
    )(hx, zn, w3, w4, b2)


def kernel(x_X, eattr_X, x_Y, eattr_Y, noise, g1_w1, g1_w2, g1_w3, g1_b,
           g2_u1, g2_u2, g2_b, w_mu, b_mu, w_lv, b_lv, w3, w4, b2,
           edge_index_X, edge_index_Y, gid_X, gid_Y):
    pad_e = EP1 - E1
    zpad = jnp.zeros((pad_e,), jnp.int32)
    trash = jnp.full((pad_e,), NPAD - 1, jnp.int32)
    srcX, dstX = edge_index_X[0], edge_index_X[1]
    srcY, dstY = edge_index_Y[0], edge_index_Y[1]
    src_xe = jnp.concatenate([srcX, zpad, srcY + N1, zpad]).reshape(-1, 128)
    src_a = jnp.concatenate([srcX, zpad, srcY + NPAD, zpad]).reshape(-1, 128)
    dst = jnp.concatenate([dstX, trash, dstY, trash]).reshape(-1, 128)
    gpad = jnp.full((NPAD - N1,), SEGP - 1, jnp.int32)
    gid_cat = jnp.concatenate([gid_X, gpad, gid_Y, gpad]).reshape(-1, 128)
    gid_zidx = jnp.concatenate(
        [gid_X, jnp.zeros((NPAD - N1,), jnp.int32)]).reshape(-1, 128)

    zeros_node = jnp.zeros((NPAD, 128), F32)
    zeros_seg = jnp.zeros((SEGP, 128), F32)
    xcat = jnp.concatenate([x_X, x_Y])
    epad = jnp.zeros((pad_e, 16), F32)
    ecat = jnp.concatenate([eattr_X, epad, eattr_Y, epad])
    noise_pad = jnp.pad(noise, ((0, 14), (0, 0)))

    xe = _mm(xcat, g1_w1, 1000)
    xs = _gather_edges(xe, src_xe)
    base, msg = _edge_base(xs, ecat, g1_w2, g1_b)
    s = None
    for it in range(3):
        s = _segsum_edges(msg, dst, zeros_node)
        if it < 2:
            a = _mm(s, g1_w3, 1024)
            g = _gather_edges(a, src_a)
            msg = _relu_add(base, g)

    h = _readout(xcat, s.reshape(2, NPAD, 128), g2_u1, g2_u2, g2_b)
    hp = h.reshape(2 * NPAD, 128)
    p = _segsum_nodes(hp, gid_cat, zeros_seg)
    z, klb = _head(p.reshape(2, SEGP, 128), noise_pad, w_mu,
                   b_mu.reshape(1, 64), w_lv, b_lv.reshape(1, 64))
    zn = _gather_z(z, gid_zidx)
    x_tilde = _decoder(hp[:N1], zn[:N1], w3, w4, b2)
    return (x_tilde, klb[0, 0])
